# baseline (speedup 1.0000x reference)
import functools

import jax
import jax.numpy as jnp
from jax import lax
from jax.experimental import pallas as pl
from jax.experimental.pallas import tpu as pltpu
from jax.experimental.pallas import tpu_sc as plsc

EPS = 1e-05
NC, NS, L = 2, 16, 16
NW = NC * NS


def _rowsums_body(h_ref, ei_ref, s_ref, q_ref, src_ref, dst_ref):
    hb = h_ref[...]
    s_ref[...] = jnp.sum(hb, axis=1)
    q_ref[...] = jnp.sum(hb * hb, axis=1)
    ei = ei_ref[...]
    src_ref[...] = ei[0]
    dst_ref[...] = ei[1]


def _norm_body(h_ref, ps_ref, pq_ref, pd_ref, gamma_ref, beta_ref, out_ref):
    hb = h_ref[...]
    d = hb.shape[1]
    ones_m = jnp.ones((8, NW), jnp.float32)
    nb_s = jax.lax.dot(ones_m, ps_ref[...],
                       preferred_element_type=jnp.float32)[0]
    nb_q = jax.lax.dot(ones_m, pq_ref[...],
                       preferred_element_type=jnp.float32)[0]
    deg = jax.lax.dot(ones_m, pd_ref[...],
                      preferred_element_type=jnp.float32)[0]
    total_sum = nb_s + jnp.sum(hb, axis=1)
    total_sq = nb_q + jnp.sum(hb * hb, axis=1)
    n_elem = (deg + 1.0) * d
    mean = total_sum / n_elem
    var = (total_sq - n_elem * mean * mean) / (n_elem - 1.0)
    std = jnp.sqrt(jnp.maximum(var, 0.0))
    has_msg = deg > 0
    mean = jnp.where(has_msg, mean, 0.0)
    std = jnp.where(has_msg, std, 0.0)
    norm_h = (hb - mean[:, None]) / (std[:, None] + EPS)
    out_ref[...] = gamma_ref[...] * norm_h + beta_ref[...]


def _make_sc_agg(n_nodes, e_per):
    mesh = plsc.VectorSubcoreMesh(core_axis_name="c", subcore_axis_name="s")
    fdt = jax.ShapeDtypeStruct((NW, n_nodes), jnp.float32)

    @functools.partial(
        pl.kernel,
        out_type=(fdt, fdt, fdt),
        mesh=mesh,
        compiler_params=pltpu.CompilerParams(needs_layout_passes=False),
        scratch_types=[
            pltpu.VMEM((n_nodes,), jnp.float32),
            pltpu.VMEM((n_nodes,), jnp.float32),
            pltpu.VMEM((n_nodes,), jnp.float32),
            pltpu.VMEM((n_nodes,), jnp.float32),
            pltpu.VMEM((n_nodes,), jnp.float32),
            pltpu.VMEM((e_per,), jnp.int32),
            pltpu.VMEM((e_per,), jnp.int32),
            pltpu.SemaphoreType.DMA,
            pltpu.SemaphoreType.DMA,
            pltpu.SemaphoreType.DMA,
            pltpu.SemaphoreType.DMA,
        ],
    )
    def sc_agg(s_hbm, q_hbm, src_hbm, dst_hbm, os_hbm, oq_hbm, od_hbm,
               s_v, q_v, acc_s, acc_q, acc_d, src_v, dst_v,
               sem0, sem1, sem2, sem3):
        wid = lax.axis_index("c") * NS + lax.axis_index("s")
        base = wid * e_per
        cp0 = pltpu.async_copy(s_hbm, s_v, sem0)
        cp1 = pltpu.async_copy(q_hbm, q_v, sem1)
        cp2 = pltpu.async_copy(src_hbm.at[pl.ds(base, e_per)], src_v, sem2)
        cp3 = pltpu.async_copy(dst_hbm.at[pl.ds(base, e_per)], dst_v, sem3)

        zeros = jnp.zeros((L,), jnp.float32)

        @plsc.parallel_loop(0, n_nodes // L, step=1, unroll=5)
        def zero_body(i):
            b = i * L
            acc_s[pl.ds(b, L)] = zeros
            acc_q[pl.ds(b, L)] = zeros
            acc_d[pl.ds(b, L)] = zeros

        cp0.wait()
        cp1.wait()
        cp2.wait()
        cp3.wait()

        ones = jnp.ones((L,), jnp.float32)

        @plsc.parallel_loop(0, e_per, step=L, unroll=25)
        def edge_body(b):
            si = src_v[pl.ds(b, L)]
            di = dst_v[pl.ds(b, L)]
            sv = plsc.load_gather(s_v, [si])
            qv = plsc.load_gather(q_v, [si])
            plsc.addupdate_scatter(acc_s, [di], sv)
            plsc.addupdate_scatter(acc_q, [di], qv)
            plsc.addupdate_scatter(acc_d, [di], ones)

        pltpu.sync_copy(acc_s, os_hbm.at[wid])
        pltpu.sync_copy(acc_q, oq_hbm.at[wid])
        pltpu.sync_copy(acc_d, od_hbm.at[wid])

    return sc_agg


def kernel(h, edge_index, gamma, beta):
    n, d = h.shape
    e = edge_index.shape[1]

    s, q, src, dst = pl.pallas_call(
        _rowsums_body,
        out_shape=(
            jax.ShapeDtypeStruct((n,), jnp.float32),
            jax.ShapeDtypeStruct((n,), jnp.float32),
            jax.ShapeDtypeStruct((e,), jnp.int32),
            jax.ShapeDtypeStruct((e,), jnp.int32),
        ),
    )(h, edge_index)

    e_per = e // NW
    ps, pq, pd = _make_sc_agg(n, e_per)(s, q, src, dst)

    out = pl.pallas_call(
        _norm_body,
        out_shape=jax.ShapeDtypeStruct((n, d), h.dtype),
    )(h, ps, pq, pd, gamma, beta)
    return out


# speedup vs baseline: 1.0449x; 1.0449x over previous
import functools

import jax
import jax.numpy as jnp
from jax import lax
from jax.experimental import pallas as pl
from jax.experimental.pallas import tpu as pltpu
from jax.experimental.pallas import tpu_sc as plsc

EPS = 1e-05
NC, NS, L = 2, 16, 16
NW = NC * NS
BG = 2048
NPAD = 5 * BG


def _rowsums_body(h_ref, ei_ref, s_ref, q_ref, src_ref, dst_ref):
    i = pl.program_id(0)
    hb = h_ref[...]
    s_ref[...] = jnp.sum(hb, axis=1)
    q_ref[...] = jnp.sum(hb * hb, axis=1)
    ei = ei_ref[...]
    eg = ei.shape[1]
    src_ref[pl.ds(i * eg, eg)] = ei[0]
    dst_ref[pl.ds(i * eg, eg)] = ei[1]


def _norm_body(h_ref, ps_ref, pq_ref, pd_ref, gamma_ref, beta_ref, out_ref):
    hb = h_ref[...]
    d = hb.shape[1]
    ones_m = jnp.ones((8, NW), jnp.float32)
    nb_s = jax.lax.dot(ones_m, ps_ref[...],
                       preferred_element_type=jnp.float32)[0]
    nb_q = jax.lax.dot(ones_m, pq_ref[...],
                       preferred_element_type=jnp.float32)[0]
    deg = jax.lax.dot(ones_m, pd_ref[...],
                      preferred_element_type=jnp.float32)[0]
    total_sum = nb_s + jnp.sum(hb, axis=1)
    total_sq = nb_q + jnp.sum(hb * hb, axis=1)
    n_elem = (deg + 1.0) * d
    mean = total_sum / n_elem
    var = (total_sq - n_elem * mean * mean) / (n_elem - 1.0)
    std = jnp.sqrt(jnp.maximum(var, 0.0))
    has_msg = deg > 0
    mean = jnp.where(has_msg, mean, 0.0)
    std = jnp.where(has_msg, std, 0.0)
    norm_h = (hb - mean[:, None]) / (std[:, None] + EPS)
    out_ref[...] = gamma_ref[...] * norm_h + beta_ref[...]


def _make_sc_agg(n_nodes, e_per):
    mesh = plsc.VectorSubcoreMesh(core_axis_name="c", subcore_axis_name="s")
    fdt = jax.ShapeDtypeStruct((NW, NPAD), jnp.float32)

    @functools.partial(
        pl.kernel,
        out_type=(fdt, fdt, fdt),
        mesh=mesh,
        compiler_params=pltpu.CompilerParams(needs_layout_passes=False),
        scratch_types=[
            pltpu.VMEM((n_nodes,), jnp.float32),
            pltpu.VMEM((n_nodes,), jnp.float32),
            pltpu.VMEM((NPAD,), jnp.float32),
            pltpu.VMEM((NPAD,), jnp.float32),
            pltpu.VMEM((NPAD,), jnp.float32),
            pltpu.VMEM((e_per,), jnp.int32),
            pltpu.VMEM((e_per,), jnp.int32),
            pltpu.SemaphoreType.DMA,
            pltpu.SemaphoreType.DMA,
            pltpu.SemaphoreType.DMA,
            pltpu.SemaphoreType.DMA,
        ],
    )
    def sc_agg(s_hbm, q_hbm, src_hbm, dst_hbm, os_hbm, oq_hbm, od_hbm,
               s_v, q_v, acc_s, acc_q, acc_d, src_v, dst_v,
               sem0, sem1, sem2, sem3):
        wid = lax.axis_index("c") * NS + lax.axis_index("s")
        base = wid * e_per
        cp0 = pltpu.async_copy(s_hbm, s_v, sem0)
        cp1 = pltpu.async_copy(q_hbm, q_v, sem1)
        cp2 = pltpu.async_copy(src_hbm.at[pl.ds(base, e_per)], src_v, sem2)
        cp3 = pltpu.async_copy(dst_hbm.at[pl.ds(base, e_per)], dst_v, sem3)

        zeros = jnp.zeros((L,), jnp.float32)

        @plsc.parallel_loop(0, NPAD // L, step=1, unroll=5)
        def zero_body(i):
            b = i * L
            acc_s[pl.ds(b, L)] = zeros
            acc_q[pl.ds(b, L)] = zeros
            acc_d[pl.ds(b, L)] = zeros

        cp0.wait()
        cp1.wait()
        cp2.wait()
        cp3.wait()

        ones = jnp.ones((L,), jnp.float32)

        @plsc.parallel_loop(0, e_per, step=L, unroll=4)
        def edge_body(b):
            si = src_v[pl.ds(b, L)]
            di = dst_v[pl.ds(b, L)]
            sv = plsc.load_gather(s_v, [si])
            qv = plsc.load_gather(q_v, [si])
            plsc.addupdate_scatter(acc_s, [di], sv)
            plsc.addupdate_scatter(acc_q, [di], qv)
            plsc.addupdate_scatter(acc_d, [di], ones)

        pltpu.sync_copy(acc_s, os_hbm.at[wid])
        pltpu.sync_copy(acc_q, oq_hbm.at[wid])
        pltpu.sync_copy(acc_d, od_hbm.at[wid])

    return sc_agg


def kernel(h, edge_index, gamma, beta):
    n, d = h.shape
    e = edge_index.shape[1]

    g = pl.cdiv(n, BG)
    eg = e // g
    s, q, src, dst = pl.pallas_call(
        _rowsums_body,
        grid=(g,),
        in_specs=[
            pl.BlockSpec((BG, d), lambda i: (i, 0)),
            pl.BlockSpec((2, eg), lambda i: (0, i)),
        ],
        out_specs=(
            pl.BlockSpec((BG,), lambda i: (i,)),
            pl.BlockSpec((BG,), lambda i: (i,)),
            pl.BlockSpec((e,), lambda i: (0,)),
            pl.BlockSpec((e,), lambda i: (0,)),
        ),
        out_shape=(
            jax.ShapeDtypeStruct((n,), jnp.float32),
            jax.ShapeDtypeStruct((n,), jnp.float32),
            jax.ShapeDtypeStruct((e,), jnp.int32),
            jax.ShapeDtypeStruct((e,), jnp.int32),
        ),
    )(h, edge_index)

    e_per = e // NW
    ps, pq, pd = _make_sc_agg(n, e_per)(s, q, src, dst)

    part_spec = pl.BlockSpec((NW, BG), lambda i: (0, i))
    vec_spec = pl.BlockSpec((d,), lambda i: (0,))
    out = pl.pallas_call(
        _norm_body,
        grid=(g,),
        in_specs=[
            pl.BlockSpec((BG, d), lambda i: (i, 0)),
            part_spec, part_spec, part_spec,
            vec_spec, vec_spec,
        ],
        out_specs=pl.BlockSpec((BG, d), lambda i: (i, 0)),
        out_shape=jax.ShapeDtypeStruct((n, d), h.dtype),
    )(h, ps, pq, pd, gamma, beta)
    return out
